# baseline (speedup 1.0000x reference)
import jax
import jax.numpy as jnp
from jax import lax
from jax.experimental import pallas as pl
from jax.experimental.pallas import tpu as pltpu

_B = 32
_STATE = 256
_HID = 1024
_F = 1000
_V = 64
_RTOT = _V * _F
_BLK = 2560
_NSPLIT = 2
_SUB = _BLK // _NSPLIT
_GRID = _RTOT // _BLK
_NSUB = _RTOT // _SUB
_LAST = _GRID - 1

_CONTRACT_LAST = (((1,), (1,)), ((), ()))


def _body(*refs):
    (s_ref, w1_ref, b1_ref, w2_ref, b2_ref, wc_ref, bc_ref) = refs[:7]
    wr_refs = refs[7:7 + _NSPLIT]
    e_ref = refs[7 + _NSPLIT]
    br_ref = refs[8 + _NSPLIT]
    cache_ref, rec_ref, h_ref, esc_ref = refs[9 + _NSPLIT:]
    i = pl.program_id(0)

    @pl.when(i == 0)
    def _trunk():
        h1 = jnp.maximum(
            lax.dot_general(s_ref[...], w1_ref[...], _CONTRACT_LAST,
                            preferred_element_type=jnp.float32) + b1_ref[...],
            0.0)
        h2 = jnp.maximum(
            lax.dot_general(h1, w2_ref[...], _CONTRACT_LAST,
                            preferred_element_type=jnp.float32) + b2_ref[...],
            0.0)
        h_ref[...] = h2

    @pl.when(i == 1)
    def _cache():
        cache_ref[...] = lax.dot_general(
            h_ref[...], wc_ref[...], _CONTRACT_LAST,
            preferred_element_type=jnp.float32) + bc_ref[...]

    @pl.when(i == 2)
    def _last_block():
        esc_ref[...] = lax.dot_general(
            h_ref[...], e_ref[...], _CONTRACT_LAST,
            preferred_element_type=jnp.float32)

    @pl.when(i < _LAST)
    def _rec():
        h = h_ref[...]
        for k in range(_NSPLIT):
            rec_ref[:, k * _SUB:(k + 1) * _SUB] = lax.dot_general(
                h, wr_refs[k][...], _CONTRACT_LAST,
                preferred_element_type=jnp.float32
            ) + br_ref[:, k * _SUB:(k + 1) * _SUB]

    @pl.when(i == _LAST)
    def _flush_last():
        rec_ref[...] = esc_ref[...] + br_ref[...]


def kernel(s, W1, b1, W2, b2, Wc, bc, Wr, br):
    b1r = b1.reshape(1, _HID)
    b2r = b2.reshape(1, _HID)
    bcr = bc.reshape(1, _F)
    brr = br.reshape(1, _RTOT)

    def wr_spec(k):
        return pl.BlockSpec(
            (_SUB, _HID),
            lambda i, k=k: (jnp.minimum(_NSPLIT * i + k,
                                        _NSUB - 2 * _NSPLIT + k), 0))

    cache, rec = pl.pallas_call(
        _body,
        grid=(_GRID,),
        in_specs=[
            pl.BlockSpec((_B, _STATE), lambda i: (0, 0)),
            pl.BlockSpec((_HID, _STATE), lambda i: (0, 0)),
            pl.BlockSpec((1, _HID), lambda i: (0, 0)),
            pl.BlockSpec((_HID, _HID), lambda i: (0, 0)),
            pl.BlockSpec((1, _HID), lambda i: (0, 0)),
            pl.BlockSpec((_F, _HID), lambda i: (0, 0)),
            pl.BlockSpec((1, _F), lambda i: (0, 0)),
        ] + [wr_spec(k) for k in range(_NSPLIT)] + [
            pl.BlockSpec((_BLK, _HID), lambda i: (_LAST, 0)),
            pl.BlockSpec((1, _BLK), lambda i: (0, i)),
        ],
        out_specs=[
            pl.BlockSpec((_B, _F), lambda i: (0, 0)),
            pl.BlockSpec((_B, _BLK), lambda i: (0, i)),
        ],
        out_shape=[
            jax.ShapeDtypeStruct((_B, _F), jnp.float32),
            jax.ShapeDtypeStruct((_B, _RTOT), jnp.float32),
        ],
        scratch_shapes=[
            pltpu.VMEM((_B, _HID), jnp.float32),
            pltpu.VMEM((_B, _BLK), jnp.float32),
        ],
        compiler_params=pltpu.CompilerParams(
            dimension_semantics=("arbitrary",)),
    )(s, W1, b1r, W2, b2r, Wc, bcr, *([Wr] * _NSPLIT), Wr, brr)

    return (cache, rec.reshape(_B, _V, _F))
